# baseline (speedup 1.0000x reference)
import jax
import jax.numpy as jnp
from jax import lax
from jax.experimental import pallas as pl

OUT_LEN = 32
_CC = 1024


def _fused_kernel(x_ref, out_ref):
    xb = x_ref[0]
    S, C = xb.shape
    RR = S // _CC

    m = jnp.max(xb, axis=-1, keepdims=True)
    z = jnp.sum(jnp.exp(xb - m), axis=-1, keepdims=True)
    lane = lax.broadcasted_iota(jnp.int32, (S, C), 1)
    pidx = jnp.min(jnp.where(xb == m, lane, C), axis=-1, keepdims=True)
    conf = (1.0 / z).reshape(RR, _CC)
    pred = pidx.astype(jnp.float32).reshape(RR, _CC)

    big = jnp.float32(S)
    posr = (lax.broadcasted_iota(jnp.int32, (RR, _CC), 0) * _CC
            + lax.broadcasted_iota(jnp.int32, (RR, _CC), 1)).astype(jnp.float32)

    last_col = pred[:, _CC - 1:]
    carry = jnp.concatenate(
        [jnp.full((1, 1), -1.0, jnp.float32), last_col[:-1]], axis=0)
    prev = jnp.concatenate([carry, pred[:, :-1]], axis=1)
    bound = pred != prev

    a = jnp.where(bound, posr, big)
    k = 1
    while k < _CC:
        a = jnp.minimum(a, jnp.concatenate(
            [a[:, k:], jnp.full((RR, k), big, jnp.float32)], axis=1))
        k *= 2
    rowmin = a[:, :1]
    k = 1
    while k < RR:
        rowmin = jnp.minimum(rowmin, jnp.concatenate(
            [rowmin[k:], jnp.full((k, 1), big, jnp.float32)], axis=0))
        k *= 2
    nextrow = jnp.concatenate(
        [rowmin[1:], jnp.full((1, 1), big, jnp.float32)], axis=0)
    sincl = jnp.minimum(a, nextrow)
    nb = jnp.concatenate(
        [sincl[:, 1:],
         jnp.concatenate([sincl[1:, :1], jnp.full((1, 1), big, jnp.float32)],
                         axis=0)], axis=1)

    voted = jnp.where(bound, conf * (nb - posr), -jnp.inf)

    col_out = lax.broadcasted_iota(jnp.int32, (1, OUT_LEN), 1)

    def body(i, state):
        v, acc = state
        mv = jnp.max(v, axis=1, keepdims=True)
        mv = jnp.max(mv, axis=0, keepdims=True)
        idx = jnp.where(v == mv, posr, big)
        idx = jnp.min(jnp.min(idx, axis=1, keepdims=True), axis=0,
                      keepdims=True)
        sel = posr == idx
        p = jnp.where(sel, pred, 0.0)
        p = jnp.max(jnp.max(p, axis=1, keepdims=True), axis=0, keepdims=True)
        val = jnp.where(mv > -jnp.inf, p, 0.0)
        acc = jnp.where(col_out == i, val, acc)
        return jnp.where(sel, -jnp.inf, v), acc

    _, acc = lax.fori_loop(
        0, OUT_LEN, body, (voted, jnp.zeros((1, OUT_LEN), jnp.float32)))
    out_ref[0] = acc


def kernel(x):
    B, S, C = x.shape

    out3 = pl.pallas_call(
        _fused_kernel,
        grid=(B,),
        in_specs=[pl.BlockSpec((1, S, C), lambda b: (b, 0, 0))],
        out_specs=pl.BlockSpec((1, 1, OUT_LEN), lambda b: (b, 0, 0)),
        out_shape=jax.ShapeDtypeStruct((B, 1, OUT_LEN), jnp.float32),
    )(x)

    return out3.reshape(B, OUT_LEN).astype(x.dtype)


# speedup vs baseline: 5.0833x; 5.0833x over previous
import jax
import jax.numpy as jnp
from jax import lax
from jax.experimental import pallas as pl
from jax.experimental.pallas import tpu as pltpu

OUT_LEN = 32


def _conf_pred_kernel(x_ref, conf_ref, pred_ref):
    xb = x_ref[0]
    C = xb.shape[-1]
    m = jnp.max(xb, axis=-1, keepdims=True)
    z = jnp.sum(jnp.exp(xb - m), axis=-1, keepdims=True)
    lane = lax.broadcasted_iota(jnp.int32, xb.shape, 1)
    pidx = jnp.min(jnp.where(xb == m, lane, C), axis=-1, keepdims=True)
    conf_ref[0] = 1.0 / z
    pred_ref[0] = pidx.astype(jnp.float32)


def _topk_kernel(conf_ref, pred_ref, out_ref, vot_ref):
    conf = conf_ref[...]
    pred = pred_ref[...]
    B, S = conf.shape
    col = lax.broadcasted_iota(jnp.int32, (B, S), 1).astype(jnp.float32)
    big = jnp.float32(S)

    prev = jnp.concatenate(
        [jnp.full((B, 1), -1.0, jnp.float32), pred[:, :-1]], axis=1)
    boundary = pred != prev

    a = jnp.where(boundary, col, big)
    nb = jnp.concatenate([a[:, 1:], jnp.full((B, 1), big, jnp.float32)],
                         axis=1)
    k = 1
    while k < S:
        nb = jnp.minimum(nb, jnp.concatenate(
            [nb[:, k:], jnp.full((B, k), big, jnp.float32)], axis=1))
        k *= 2

    vot_ref[...] = jnp.where(boundary, conf * (nb - col), -jnp.inf)

    col_out = lax.broadcasted_iota(jnp.int32, (1, OUT_LEN), 1)

    def body(i, acc):
        v = vot_ref[...]
        mv = jnp.max(v, axis=1, keepdims=True)
        idx = jnp.min(jnp.where(v == mv, col, big), axis=1, keepdims=True)
        sel = col == idx
        p = jnp.max(jnp.where(sel, pred, 0.0), axis=1, keepdims=True)
        val = jnp.where(mv > -jnp.inf, p, 0.0)
        vot_ref[...] = jnp.where(sel, -jnp.inf, v)
        return jnp.where(col_out == i, val, acc)

    out_ref[...] = lax.fori_loop(0, OUT_LEN, body,
                                 jnp.zeros((B, OUT_LEN), jnp.float32))


def kernel(x):
    B, S, C = x.shape
    R = S

    conf3, pred3 = pl.pallas_call(
        _conf_pred_kernel,
        grid=(B, S // R),
        in_specs=[pl.BlockSpec((1, R, C), lambda b, s: (b, s, 0))],
        out_specs=[
            pl.BlockSpec((1, R, 1), lambda b, s: (b, s, 0)),
            pl.BlockSpec((1, R, 1), lambda b, s: (b, s, 0)),
        ],
        out_shape=[
            jax.ShapeDtypeStruct((B, S, 1), jnp.float32),
            jax.ShapeDtypeStruct((B, S, 1), jnp.float32),
        ],
    )(x)

    out = pl.pallas_call(
        _topk_kernel,
        out_shape=jax.ShapeDtypeStruct((B, OUT_LEN), jnp.float32),
        scratch_shapes=[pltpu.VMEM((B, S), jnp.float32)],
    )(conf3.reshape(B, S), pred3.reshape(B, S))

    return out.astype(x.dtype)


# speedup vs baseline: 5.1917x; 1.0213x over previous
import jax
import jax.numpy as jnp
from jax import lax
from jax.experimental import pallas as pl
from jax.experimental.pallas import tpu as pltpu

OUT_LEN = 32


def _conf_pred_kernel(x_ref, conf_ref, pred_ref):
    xb = x_ref[0]
    C = xb.shape[-1]
    m = jnp.max(xb, axis=-1, keepdims=True)
    z = jnp.sum(jnp.exp(xb - m), axis=-1, keepdims=True)
    lane = lax.broadcasted_iota(jnp.int32, xb.shape, 1)
    pidx = jnp.min(jnp.where(xb == m, lane, C), axis=-1, keepdims=True)
    conf_ref[0] = 1.0 / z
    pred_ref[0] = pidx.astype(jnp.float32)


def _topk_kernel(conf_ref, pred_ref, out_ref, vot_ref, pay_ref):
    conf = conf_ref[...]
    pred = pred_ref[...]
    B, S = conf.shape
    col = lax.broadcasted_iota(jnp.int32, (B, S), 1).astype(jnp.float32)
    big = jnp.float32(S)
    paybig = jnp.float32(1 << 22)

    prev = jnp.concatenate(
        [jnp.full((B, 1), -1.0, jnp.float32), pred[:, :-1]], axis=1)
    boundary = pred != prev

    a = jnp.where(boundary, col, big)
    nb = jnp.concatenate([a[:, 1:], jnp.full((B, 1), big, jnp.float32)],
                         axis=1)
    k = 1
    while k < S:
        nb = jnp.minimum(nb, jnp.concatenate(
            [nb[:, k:], jnp.full((B, k), big, jnp.float32)], axis=1))
        k *= 2

    vot_ref[...] = jnp.where(boundary, conf * (nb - col), -jnp.inf)
    pay_ref[...] = col * 256.0 + pred

    col_out = lax.broadcasted_iota(jnp.int32, (1, OUT_LEN), 1)

    def body(i, acc):
        v = vot_ref[...]
        pay = pay_ref[...]
        mv = jnp.max(v, axis=1, keepdims=True)
        psel = jnp.min(jnp.where(v == mv, pay, paybig), axis=1,
                       keepdims=True)
        pos = jnp.floor(psel * (1.0 / 256.0))
        p = psel - pos * 256.0
        val = jnp.where(mv > -jnp.inf, p, 0.0)
        vot_ref[...] = jnp.where(pay == psel, -jnp.inf, v)
        return jnp.where(col_out == i, val, acc)

    out_ref[...] = lax.fori_loop(0, OUT_LEN, body,
                                 jnp.zeros((B, OUT_LEN), jnp.float32))


def kernel(x):
    B, S, C = x.shape
    R = S

    conf3, pred3 = pl.pallas_call(
        _conf_pred_kernel,
        grid=(B, S // R),
        in_specs=[pl.BlockSpec((1, R, C), lambda b, s: (b, s, 0))],
        out_specs=[
            pl.BlockSpec((1, R, 1), lambda b, s: (b, s, 0)),
            pl.BlockSpec((1, R, 1), lambda b, s: (b, s, 0)),
        ],
        out_shape=[
            jax.ShapeDtypeStruct((B, S, 1), jnp.float32),
            jax.ShapeDtypeStruct((B, S, 1), jnp.float32),
        ],
    )(x)

    out = pl.pallas_call(
        _topk_kernel,
        out_shape=jax.ShapeDtypeStruct((B, OUT_LEN), jnp.float32),
        scratch_shapes=[pltpu.VMEM((B, S), jnp.float32),
                        pltpu.VMEM((B, S), jnp.float32)],
    )(conf3.reshape(B, S), pred3.reshape(B, S))

    return out.astype(x.dtype)


# speedup vs baseline: 5.2420x; 1.0097x over previous
import jax
import jax.numpy as jnp
from jax import lax
from jax.experimental import pallas as pl
from jax.experimental.pallas import tpu as pltpu

OUT_LEN = 32


def _conf_pred_kernel(x_ref, conf_ref, pred_ref):
    xb = x_ref[0]
    C = xb.shape[-1]
    m = jnp.max(xb, axis=-1, keepdims=True)
    z = jnp.sum(jnp.exp(xb - m), axis=-1, keepdims=True)
    lane = lax.broadcasted_iota(jnp.int32, xb.shape, 1)
    pidx = jnp.min(jnp.where(xb == m, lane, C), axis=-1, keepdims=True)
    conf_ref[0] = 1.0 / z
    pred_ref[0] = pidx.astype(jnp.float32)


def _topk_kernel(conf_ref, pred_ref, out_ref, vot_ref, pay_ref):
    conf = conf_ref[...]
    pred = pred_ref[...]
    B, S = conf.shape
    col = lax.broadcasted_iota(jnp.int32, (B, S), 1).astype(jnp.float32)
    big = jnp.float32(S)
    paybig = jnp.float32(1 << 22)

    prev = jnp.concatenate(
        [jnp.full((B, 1), -1.0, jnp.float32), pred[:, :-1]], axis=1)
    boundary = pred != prev

    a = jnp.where(boundary, col, big)
    nb = jnp.concatenate([a[:, 1:], jnp.full((B, 1), big, jnp.float32)],
                         axis=1)
    k = 1
    while k < S:
        nb = jnp.minimum(nb, jnp.concatenate(
            [nb[:, k:], jnp.full((B, k), big, jnp.float32)], axis=1))
        k *= 2

    vot_ref[...] = jnp.where(boundary, conf * (nb - col), -jnp.inf)
    pay_ref[...] = col * 256.0 + pred

    col_out = lax.broadcasted_iota(jnp.int32, (1, OUT_LEN), 1)

    def body(i, acc):
        v = vot_ref[...]
        pay = pay_ref[...]
        mv = jnp.max(v, axis=1, keepdims=True)
        psel = jnp.min(jnp.where(v == mv, pay, paybig), axis=1,
                       keepdims=True)
        pos = jnp.floor(psel * (1.0 / 256.0))
        p = psel - pos * 256.0
        val = jnp.where(mv > -jnp.inf, p, 0.0)
        vot_ref[...] = jnp.where(pay == psel, -jnp.inf, v)
        return jnp.where(col_out == i, val, acc)

    acc = jnp.zeros((B, OUT_LEN), jnp.float32)
    for i in range(OUT_LEN):
        acc = body(i, acc)
    out_ref[...] = acc


def kernel(x):
    B, S, C = x.shape
    R = S

    conf3, pred3 = pl.pallas_call(
        _conf_pred_kernel,
        grid=(B, S // R),
        in_specs=[pl.BlockSpec((1, R, C), lambda b, s: (b, s, 0))],
        out_specs=[
            pl.BlockSpec((1, R, 1), lambda b, s: (b, s, 0)),
            pl.BlockSpec((1, R, 1), lambda b, s: (b, s, 0)),
        ],
        out_shape=[
            jax.ShapeDtypeStruct((B, S, 1), jnp.float32),
            jax.ShapeDtypeStruct((B, S, 1), jnp.float32),
        ],
    )(x)

    out = pl.pallas_call(
        _topk_kernel,
        out_shape=jax.ShapeDtypeStruct((B, OUT_LEN), jnp.float32),
        scratch_shapes=[pltpu.VMEM((B, S), jnp.float32),
                        pltpu.VMEM((B, S), jnp.float32)],
    )(conf3.reshape(B, S), pred3.reshape(B, S))

    return out.astype(x.dtype)
